# baseline (speedup 1.0000x reference)
import functools

import jax
import jax.numpy as jnp
from jax import lax
from jax.experimental import pallas as pl
from jax.experimental.pallas import tpu as pltpu
from jax.experimental.pallas import tpu_sc as plsc

N = 4096
NW = 32
L = 16
S = 2048
RPW = N // NW
CH = 8

_mesh = plsc.VectorSubcoreMesh(core_axis_name="c", subcore_axis_name="s")
_cparams = pltpu.CompilerParams(needs_layout_passes=False)


@functools.partial(
    pl.kernel,
    out_type=jax.ShapeDtypeStruct((S,), jnp.float32),
    mesh=_mesh,
    compiler_params=_cparams,
    scratch_types=[
        pltpu.VMEM((CH, N), jnp.float32),
        pltpu.VMEM((CH, N), jnp.float32),
        pltpu.VMEM((S // NW,), jnp.float32),
        pltpu.SemaphoreType.DMA,
        pltpu.SemaphoreType.DMA,
    ],
)
def _sc_rowsum(mat_hbm, rs_hbm, buf0, buf1, rs_v, sem0, sem1):
    rpw = S // NW
    nch = rpw // CH
    wid = lax.axis_index("s") * 2 + lax.axis_index("c")
    base = wid * rpw
    bufs = (buf0, buf1)
    sems = (sem0, sem1)
    zero = jnp.zeros((L,), jnp.float32)
    lanes = lax.iota(jnp.int32, L)

    cps = [None, None]
    cps[0] = pltpu.async_copy(mat_hbm.at[pl.ds(base, CH)], buf0, sem0)
    for k in range(nch):
        if k + 1 < nch:
            nb = (k + 1) % 2
            cps[nb] = pltpu.async_copy(
                mat_hbm.at[pl.ds(base + (k + 1) * CH, CH)], bufs[nb], sems[nb]
            )
        cps[k % 2].wait()
        buf = bufs[k % 2]
        g16 = (k // 2) * L
        lane0 = (k % 2) * CH

        def row_body(r, _, buf=buf, g16=g16, lane0=lane0):
            def col_body(j, accs):
                a0, a1, a2, a3 = accs
                c0 = pl.multiple_of(j * 256, 256)
                for q in range(4):
                    o = c0 + q * 64
                    a0 = a0 + buf[r, pl.ds(o, L)]
                    a1 = a1 + buf[r, pl.ds(o + 16, L)]
                    a2 = a2 + buf[r, pl.ds(o + 32, L)]
                    a3 = a3 + buf[r, pl.ds(o + 48, L)]
                return (a0, a1, a2, a3)

            a0, a1, a2, a3 = lax.fori_loop(
                0, 16, col_body, (zero, zero, zero, zero)
            )
            s = jnp.sum((a0 + a1) + (a2 + a3))
            old = rs_v[pl.ds(g16, L)]
            rs_v[pl.ds(g16, L)] = jnp.where(lanes == lane0 + r, s, old)
            return 0

        lax.fori_loop(0, CH, row_body, 0)

    pltpu.sync_copy(rs_v, rs_hbm.at[pl.ds(base, rpw)])


_TCB = 256


def _tc_body(x_ref, o_ref):
    o_ref[...] = jnp.sum(x_ref[...], axis=1)


def _tc_rowsum(mat):
    r = N - S
    return pl.pallas_call(
        _tc_body,
        grid=(r // _TCB,),
        in_specs=[pl.BlockSpec((_TCB, N), lambda i: (S // _TCB + i, 0))],
        out_specs=pl.BlockSpec((_TCB,), lambda i: (i,)),
        out_shape=jax.ShapeDtypeStruct((r,), jnp.float32),
    )(mat)


@functools.partial(
    pl.kernel,
    out_type=jax.ShapeDtypeStruct((N - 1,), jnp.float32),
    mesh=_mesh,
    compiler_params=_cparams,
    scratch_types=[
        pltpu.VMEM((N,), jnp.float32),
        pltpu.VMEM((RPW,), jnp.int32),
        pltpu.VMEM((RPW,), jnp.float32),
        pltpu.VMEM((L,), jnp.int32),
        pltpu.SemaphoreType.DMA,
    ],
)
def _sc_compact(last_hbm, rslo_hbm, rshi_hbm, out_hbm, rs_v, idx_v, val_v,
                last_v, sem):
    wid = lax.axis_index("s") * 2 + lax.axis_index("c")
    base = wid * RPW
    pltpu.sync_copy(rslo_hbm, rs_v.at[pl.ds(0, S)])
    pltpu.sync_copy(rshi_hbm, rs_v.at[pl.ds(S, N - S)])
    pltpu.sync_copy(last_hbm, last_v)
    last_vec = last_v[...]
    lanes = lax.iota(jnp.int32, L)
    zi = jnp.zeros((L,), jnp.int32)

    def load_masked(i):
        v = rs_v[pl.ds(pl.multiple_of(i * L, L), L)]
        return jnp.where(lanes + i * L == last_vec, -1.0, v)

    def cnt_body(i, carry):
        off_acc, k_acc = carry
        mi = jnp.where(load_masked(i) != -1.0, 1, 0).astype(jnp.int32)
        off_acc = jnp.where(i * L < base, off_acc + mi, off_acc)
        return (off_acc, k_acc + mi)

    off_acc, k_acc = lax.fori_loop(0, N // L, cnt_body, (zi, zi))
    off = jnp.sum(off_acc)
    kk = jnp.sum(k_acc)
    dd = N - kk
    dp = base - off

    def fk_body(g, carry):
        kr, fk = carry
        v = load_masked(base // L + g)
        m = v != -1.0
        mi = jnp.where(m, 1, 0).astype(jnp.int32)
        ck = plsc.cumsum(mi)
        is_fk = m & (ck == 1) & (kr == 0)
        fk = fk + jnp.sum(jnp.where(is_fk, v, 0.0))
        return (kr + jnp.sum(mi), fk)

    _, fk = lax.fori_loop(0, RPW // L, fk_body,
                          (jnp.int32(0), jnp.float32(0.0)))

    def pos_body(g, carry):
        kr, dr = carry
        i = base // L + g
        v = load_masked(i)
        m = v != -1.0
        mi = jnp.where(m, 1, 0).astype(jnp.int32)
        ck = plsc.cumsum(mi)
        cd = plsc.cumsum(1 - mi)
        kpos = ((off + kr) + ck) - 1
        dpos = ((kk + dp) + dr) + cd - 1
        ovf = (~m) & (dpos == N - 1)
        redirect = jnp.where(dd >= 2, kk, off)
        pos = jnp.where(m, kpos, jnp.where(ovf, redirect, dpos))
        vv = jnp.where(ovf & (dd == 1), fk, v)
        g16 = pl.multiple_of(g * L, L)
        idx_v[pl.ds(g16, L)] = pos
        val_v[pl.ds(g16, L)] = vv
        return (kr + jnp.sum(mi), dr + jnp.sum(1 - mi))

    lax.fori_loop(0, RPW // L, pos_body, (jnp.int32(0), jnp.int32(0)))
    pltpu.async_copy(val_v, out_hbm.at[idx_v], sem).wait()


def kernel(first_cam_trap, last_cam_trap, cond_mat):
    last16 = jnp.broadcast_to(last_cam_trap.astype(jnp.int32), (L,))
    rs_lo = _sc_rowsum(cond_mat)
    rs_hi = _tc_rowsum(cond_mat)
    out = _sc_compact(last16, rs_lo, rs_hi)
    return out.reshape(-1, 1)


# speedup vs baseline: 2.0987x; 2.0987x over previous
import functools

import jax
import jax.numpy as jnp
from jax import lax
from jax.experimental import pallas as pl
from jax.experimental.pallas import tpu as pltpu
from jax.experimental.pallas import tpu_sc as plsc

N = 4096
NW = 32
L = 16
RPW = N // NW
_TCB = 512

_mesh = plsc.VectorSubcoreMesh(core_axis_name="c", subcore_axis_name="s")
_cparams = pltpu.CompilerParams(needs_layout_passes=False)


def _tc_body(x_ref, last_ref, o_ref, l16_ref):
    o_ref[...] = jnp.sum(x_ref[...], axis=1)

    @pl.when(pl.program_id(0) == 0)
    def _():
        l16_ref[...] = jnp.broadcast_to(last_ref[0], (128,)).astype(jnp.int32)


def _tc_rowsum(mat, last):
    return pl.pallas_call(
        _tc_body,
        grid=(N // _TCB,),
        in_specs=[
            pl.BlockSpec((_TCB, N), lambda i: (i, 0)),
            pl.BlockSpec(memory_space=pltpu.SMEM),
        ],
        out_specs=[
            pl.BlockSpec((_TCB,), lambda i: (i,)),
            pl.BlockSpec((128,), lambda i: (0,)),
        ],
        out_shape=[
            jax.ShapeDtypeStruct((N,), jnp.float32),
            jax.ShapeDtypeStruct((128,), jnp.int32),
        ],
    )(mat, last)


@functools.partial(
    pl.kernel,
    out_type=jax.ShapeDtypeStruct((N,), jnp.float32),
    mesh=_mesh,
    compiler_params=_cparams,
    scratch_types=[
        pltpu.VMEM((N,), jnp.float32),
        pltpu.VMEM((RPW,), jnp.float32),
        pltpu.VMEM((L,), jnp.int32),
        pltpu.SemaphoreType.DMA,
    ],
)
def _sc_compact(last_hbm, rs_hbm, out_hbm, rs_v, buf_v, last_v, sem):
    wid = lax.axis_index("s") * 2 + lax.axis_index("c")
    lo = wid * RPW
    hi = lo + RPW
    cp = pltpu.async_copy(rs_hbm, rs_v, sem)
    pltpu.sync_copy(last_hbm.at[pl.ds(0, L)], last_v)
    last_vec = last_v[...]
    lanes = lax.iota(jnp.int32, L)
    neg1 = jnp.full((L,), -1.0, jnp.float32)
    ones = jnp.ones((L,), jnp.int32)

    for g in range(RPW // L):
        buf_v[pl.ds(g * L, L)] = neg1
    cp.wait()

    i0 = pl.multiple_of((last_vec[0] // L) * L, L)
    v0 = rs_v[pl.ds(i0, L)]
    rs_v[pl.ds(i0, L)] = jnp.where(lanes + i0 == last_vec, -1.0, v0)

    def count4(j, prefix):
        for q in range(4):
            o = pl.multiple_of(j * (4 * L) + q * L, L)
            m = rs_v[pl.ds(o, L)] != -1.0
            prefix = prefix + plsc.all_reduce_population_count(m)
        return prefix

    prefix0 = lax.fori_loop(0, wid * 2, count4, jnp.zeros((L,), jnp.int32))

    def cond(c):
        i, prefix = c
        return (i < N // L) & jnp.any(prefix < hi)

    def body(c):
        i, prefix = c
        v = rs_v[pl.ds(pl.multiple_of(i * L, L), L)]
        m = v != -1.0
        cnt = plsc.all_reduce_population_count(m)
        rank0 = (prefix + plsc.cumsum(ones, mask=m)) - 1
        sm = m & (rank0 >= lo) & (rank0 < hi)
        plsc.store_scatter(buf_v, [rank0 - lo], v, mask=sm)
        return (i + 1, prefix + cnt)

    lax.while_loop(cond, body, (wid * (RPW // L), prefix0))
    pltpu.sync_copy(buf_v, out_hbm.at[pl.ds(lo, RPW)])


def kernel(first_cam_trap, last_cam_trap, cond_mat):
    rs, last128 = _tc_rowsum(cond_mat, last_cam_trap.astype(jnp.int32))
    out = _sc_compact(last128, rs)
    return out[: N - 1].reshape(-1, 1)


# speedup vs baseline: 2.1117x; 1.0062x over previous
import functools

import jax
import jax.numpy as jnp
from jax import lax
from jax.experimental import pallas as pl
from jax.experimental.pallas import tpu as pltpu
from jax.experimental.pallas import tpu_sc as plsc

N = 4096
NW = 32
L = 16
RPW = N // NW
_TCB = 512

_mesh = plsc.VectorSubcoreMesh(core_axis_name="c", subcore_axis_name="s")
_cparams = pltpu.CompilerParams(needs_layout_passes=False)


def _tc_body(x_ref, last_ref, o_ref, l16_ref):
    o_ref[...] = jnp.sum(x_ref[...], axis=1)

    @pl.when(pl.program_id(0) == 0)
    def _():
        l16_ref[...] = jnp.broadcast_to(last_ref[0], (128,)).astype(jnp.int32)


def _tc_rowsum(mat, last):
    return pl.pallas_call(
        _tc_body,
        grid=(N // _TCB,),
        in_specs=[
            pl.BlockSpec((_TCB, N), lambda i: (i, 0)),
            pl.BlockSpec(memory_space=pltpu.SMEM),
        ],
        out_specs=[
            pl.BlockSpec((_TCB,), lambda i: (i,)),
            pl.BlockSpec((128,), lambda i: (0,)),
        ],
        out_shape=[
            jax.ShapeDtypeStruct((N,), jnp.float32),
            jax.ShapeDtypeStruct((128,), jnp.int32),
        ],
    )(mat, last)


@functools.partial(
    pl.kernel,
    out_type=jax.ShapeDtypeStruct((N,), jnp.float32),
    mesh=_mesh,
    compiler_params=_cparams,
    scratch_types=[
        pltpu.VMEM((N,), jnp.float32),
        pltpu.VMEM((RPW,), jnp.float32),
        pltpu.VMEM((L,), jnp.int32),
        pltpu.SemaphoreType.DMA,
    ],
)
def _sc_compact(last_hbm, rs_hbm, out_hbm, rs_v, buf_v, last_v, sem):
    wid = lax.axis_index("s") * 2 + lax.axis_index("c")
    lo = wid * RPW
    hi = lo + RPW
    cp = pltpu.async_copy(rs_hbm, rs_v, sem)
    pltpu.sync_copy(last_hbm.at[pl.ds(0, L)], last_v)
    last_vec = last_v[...]
    lanes = lax.iota(jnp.int32, L)
    neg1 = jnp.full((L,), -1.0, jnp.float32)
    ones = jnp.ones((L,), jnp.int32)

    for g in range(RPW // L):
        buf_v[pl.ds(g * L, L)] = neg1
    cp.wait()

    i0 = pl.multiple_of((last_vec[0] // L) * L, L)
    v0 = rs_v[pl.ds(i0, L)]
    rs_v[pl.ds(i0, L)] = jnp.where(lanes + i0 == last_vec, -1.0, v0)

    def count8(j, carry):
        p0, p1 = carry
        for q in range(4):
            o = pl.multiple_of(j * (8 * L) + q * (2 * L), L)
            m0 = rs_v[pl.ds(o, L)] != -1.0
            m1 = rs_v[pl.ds(o + L, L)] != -1.0
            p0 = p0 + plsc.all_reduce_population_count(m0)
            p1 = p1 + plsc.all_reduce_population_count(m1)
        return (p0, p1)

    zi = jnp.zeros((L,), jnp.int32)
    p0, p1 = lax.fori_loop(0, wid, count8, (zi, zi))
    prefix0 = p0 + p1

    def cond(c):
        i, prefix = c
        return (i < N // L) & jnp.any(prefix < hi)

    def body(c):
        i, prefix = c
        v = rs_v[pl.ds(pl.multiple_of(i * L, L), L)]
        m = v != -1.0
        cnt = plsc.all_reduce_population_count(m)
        rank0 = (prefix + plsc.cumsum(ones, mask=m)) - 1
        sm = m & (rank0 >= lo) & (rank0 < hi)
        plsc.store_scatter(buf_v, [rank0 - lo], v, mask=sm)
        return (i + 1, prefix + cnt)

    lax.while_loop(cond, body, (wid * (RPW // L), prefix0))
    pltpu.sync_copy(buf_v, out_hbm.at[pl.ds(lo, RPW)])


def kernel(first_cam_trap, last_cam_trap, cond_mat):
    rs, last128 = _tc_rowsum(cond_mat, last_cam_trap.astype(jnp.int32))
    out = _sc_compact(last128, rs)
    return out[: N - 1].reshape(-1, 1)
